# baseline (speedup 1.0000x reference)
import functools

import jax
import jax.numpy as jnp
from jax import lax
from jax.experimental import pallas as pl
from jax.experimental.pallas import tpu as pltpu
from jax.experimental.pallas import tpu_sc as plsc

VOCAB = 1000000
EMB = 64
BATCH = 4096
SEQ = 50
N = BATCH * SEQ

NUM_CORES = 2
NUM_SUBCORES = 16
NW = NUM_CORES * NUM_SUBCORES
PER_W = N // NW
CHUNK = 640
NCHUNK = PER_W // CHUNK
NBUF = 3


def _make_sc_gather():
    mesh = plsc.VectorSubcoreMesh(core_axis_name="c", subcore_axis_name="s")

    @functools.partial(
        pl.kernel,
        mesh=mesh,
        out_type=jax.ShapeDtypeStruct((N, EMB), jnp.float32),
        scratch_types=[
            pltpu.VMEM((PER_W,), jnp.int32),
            pltpu.VMEM((NBUF, CHUNK, EMB), jnp.float32),
            pltpu.SemaphoreType.DMA((NBUF,)),
            pltpu.SemaphoreType.DMA((NBUF,)),
        ],
        compiler_params=pltpu.CompilerParams(use_tc_tiling_on_sc=False),
    )
    def k(idx_hbm, table_hbm, out_hbm, idx_v, rows_v, gsem, wsem):
        wid = lax.axis_index("s") * NUM_CORES + lax.axis_index("c")
        base = pl.multiple_of(wid * PER_W, PER_W)
        pltpu.sync_copy(idx_hbm.at[pl.ds(base, PER_W)], idx_v)

        def gather_start(g, b):
            return pltpu.async_copy(
                table_hbm.at[idx_v.at[pl.ds(g * CHUNK, CHUNK)]],
                rows_v.at[b],
                gsem.at[b],
            )

        ghandle = [None] * NBUF
        whandle = [None] * NBUF
        for g in range(min(NBUF - 1, NCHUNK)):
            ghandle[g % NBUF] = gather_start(g, g % NBUF)
        for g in range(NCHUNK):
            b = g % NBUF
            ng = g + NBUF - 1
            if ng < NCHUNK:
                nb = ng % NBUF
                if whandle[nb] is not None:
                    whandle[nb].wait()
                ghandle[nb] = gather_start(ng, nb)
            ghandle[b].wait()
            whandle[b] = pltpu.async_copy(
                rows_v.at[b],
                out_hbm.at[pl.ds(base + g * CHUNK, CHUNK)],
                wsem.at[b],
            )
        for b in range(NBUF):
            if whandle[b] is not None:
                whandle[b].wait()

    return k


_sc_gather = _make_sc_gather()


def kernel(batch_sentence1, table):
    idx_flat = batch_sentence1.reshape(N).astype(jnp.int32)
    out = _sc_gather(idx_flat, table)
    return out.reshape(BATCH, SEQ, EMB)


# speedup vs baseline: 1.2879x; 1.2879x over previous
import functools

import jax
import jax.numpy as jnp
from jax import lax
from jax.experimental import pallas as pl
from jax.experimental.pallas import tpu as pltpu
from jax.experimental.pallas import tpu_sc as plsc

VOCAB = 1000000
EMB = 64
BATCH = 4096
SEQ = 50
N = BATCH * SEQ

NUM_CORES = 2
NUM_SUBCORES = 16
NW = NUM_CORES * NUM_SUBCORES
PER_W = N // NW
CHUNK = 640
NCHUNK = PER_W // CHUNK
NBUF = 3


def _make_sc_gather():
    mesh = plsc.VectorSubcoreMesh(core_axis_name="c", subcore_axis_name="s")

    @functools.partial(
        pl.kernel,
        mesh=mesh,
        out_type=jax.ShapeDtypeStruct((N, EMB), jnp.float32),
        scratch_types=[
            pltpu.VMEM((PER_W,), jnp.int32),
            pltpu.VMEM((NBUF, CHUNK, EMB), jnp.float32),
            pltpu.SemaphoreType.DMA((NBUF,)),
            pltpu.SemaphoreType.DMA((NBUF,)),
        ],
        compiler_params=pltpu.CompilerParams(use_tc_tiling_on_sc=False),
    )
    def k(idx_hbm, table_hbm, out_hbm, idx_v, rows_v, gsem, wsem):
        wid = lax.axis_index("s") * NUM_CORES + lax.axis_index("c")
        base = pl.multiple_of(wid * PER_W, PER_W)
        pltpu.sync_copy(idx_hbm.at[pl.ds(base, PER_W)], idx_v)

        def gather_start(g, b):
            return pltpu.async_copy(
                table_hbm.at[idx_v.at[pl.ds(g * CHUNK, CHUNK)]],
                rows_v.at[b],
                gsem.at[b],
            )

        ghandle = [None] * NBUF
        whandle = [None] * NBUF
        for g in range(min(NBUF - 1, NCHUNK)):
            ghandle[g % NBUF] = gather_start(g, g % NBUF)
        for g in range(NCHUNK):
            b = g % NBUF
            ng = g + NBUF - 1
            if ng < NCHUNK:
                nb = ng % NBUF
                if whandle[nb] is not None:
                    whandle[nb].wait()
                ghandle[nb] = gather_start(ng, nb)
            ghandle[b].wait()
            whandle[b] = pltpu.async_copy(
                rows_v.at[b],
                out_hbm.at[pl.ds(base + g * CHUNK, CHUNK)],
                wsem.at[b],
            )
        for b in range(NBUF):
            if whandle[b] is not None:
                whandle[b].wait()

    return k


_sc_gather = _make_sc_gather()


TBLOCK = 4096
TSTEPS = -(-VOCAB // TBLOCK)


def _tc_relayout(tT):
    def body(in_ref, out_ref):
        y = in_ref[...].T.reshape(TBLOCK // 2, 2, EMB)
        out_ref[...] = jnp.concatenate([y[:, 0, :], y[:, 1, :]], axis=1)

    return pl.pallas_call(
        body,
        grid=(TSTEPS,),
        in_specs=[pl.BlockSpec((EMB, TBLOCK), lambda i: (0, i))],
        out_specs=pl.BlockSpec((TBLOCK // 2, 2 * EMB), lambda i: (i, 0)),
        out_shape=jax.ShapeDtypeStruct((VOCAB // 2, 2 * EMB), jnp.float32),
    )(tT)


def kernel(batch_sentence1, table):
    idx_flat = batch_sentence1.reshape(N).astype(jnp.int32)
    table_lin = _tc_relayout(table.T).reshape(VOCAB, EMB)
    out = _sc_gather(idx_flat, table_lin)
    return out.reshape(BATCH, SEQ, EMB)


# speedup vs baseline: 1.3803x; 1.0717x over previous
import functools

import jax
import jax.numpy as jnp
from jax import lax
from jax.experimental import pallas as pl
from jax.experimental.pallas import tpu as pltpu
from jax.experimental.pallas import tpu_sc as plsc

VOCAB = 1000000
EMB = 64
BATCH = 4096
SEQ = 50
N = BATCH * SEQ

NUM_CORES = 2
NUM_SUBCORES = 16
NW = NUM_CORES * NUM_SUBCORES
PER_W = N // NW
CHUNK = 640
NCHUNK = PER_W // CHUNK
NBUF = 3


def _make_sc_gather():
    mesh = plsc.VectorSubcoreMesh(core_axis_name="c", subcore_axis_name="s")

    @functools.partial(
        pl.kernel,
        mesh=mesh,
        out_type=jax.ShapeDtypeStruct((N, EMB), jnp.float32),
        scratch_types=[
            pltpu.VMEM((PER_W,), jnp.int32),
            pltpu.VMEM((NBUF, CHUNK, EMB), jnp.float32),
            pltpu.SemaphoreType.DMA((NBUF,)),
            pltpu.SemaphoreType.DMA((NBUF,)),
        ],
        compiler_params=pltpu.CompilerParams(use_tc_tiling_on_sc=False),
    )
    def k(idx_hbm, table_hbm, out_hbm, idx_v, rows_v, gsem, wsem):
        wid = lax.axis_index("s") * NUM_CORES + lax.axis_index("c")
        base = pl.multiple_of(wid * PER_W, PER_W)
        pltpu.sync_copy(idx_hbm.at[pl.ds(base, PER_W)], idx_v)

        def gather_start(g, b):
            return pltpu.async_copy(
                table_hbm.at[idx_v.at[pl.ds(g * CHUNK, CHUNK)]],
                rows_v.at[b],
                gsem.at[b],
            )

        ghandle = [None] * NBUF
        whandle = [None] * NBUF
        for g in range(min(NBUF - 1, NCHUNK)):
            ghandle[g % NBUF] = gather_start(g, g % NBUF)
        for g in range(NCHUNK):
            b = g % NBUF
            ng = g + NBUF - 1
            if ng < NCHUNK:
                nb = ng % NBUF
                if whandle[nb] is not None:
                    whandle[nb].wait()
                ghandle[nb] = gather_start(ng, nb)
            ghandle[b].wait()
            whandle[b] = pltpu.async_copy(
                rows_v.at[b],
                out_hbm.at[pl.ds(base + g * CHUNK, CHUNK)],
                wsem.at[b],
            )
        for b in range(NBUF):
            if whandle[b] is not None:
                whandle[b].wait()

    return k


_sc_gather = _make_sc_gather()


TBLOCK = 4096
TSTEPS = -(-VOCAB // TBLOCK)


def _tc_relayout(tT):
    def body(in_ref, out_ref):
        y = in_ref[...].T.reshape(TBLOCK // 2, 2, EMB)
        out_ref[...] = jnp.concatenate([y[:, 0, :], y[:, 1, :]], axis=1)

    return pl.pallas_call(
        body,
        grid=(TSTEPS,),
        in_specs=[pl.BlockSpec((EMB, TBLOCK), lambda i: (0, i))],
        out_specs=pl.BlockSpec((TBLOCK // 2, 2 * EMB), lambda i: (i, 0)),
        out_shape=jax.ShapeDtypeStruct((VOCAB // 2, 2 * EMB), jnp.float32),
    )(tT)


OBATCH = 256


def _tc_outlayout(x2):
    def body(in_ref, out_ref):
        out_ref[...] = in_ref[...].T

    return pl.pallas_call(
        body,
        grid=(BATCH // OBATCH,),
        in_specs=[pl.BlockSpec((OBATCH, SEQ * EMB), lambda i: (i, 0))],
        out_specs=pl.BlockSpec((SEQ * EMB, OBATCH), lambda i: (0, i)),
        out_shape=jax.ShapeDtypeStruct((SEQ * EMB, BATCH), jnp.float32),
    )(x2)


def kernel(batch_sentence1, table):
    idx_flat = batch_sentence1.reshape(N).astype(jnp.int32)
    table_lin = _tc_relayout(table.T).reshape(VOCAB, EMB)
    out = _sc_gather(idx_flat, table_lin)
    out_t = _tc_outlayout(out.reshape(BATCH, SEQ * EMB))
    return out_t.reshape(SEQ, EMB, BATCH).transpose(2, 0, 1)
